# baseline (speedup 1.0000x reference)
import jax
import jax.numpy as jnp
from jax import lax
from jax.experimental import pallas as pl
from jax.experimental.pallas import tpu as pltpu
from jax.experimental.pallas import tpu_sc as plsc


def _scale_body(logits_ref, temp_ref, out_ref):
    inv = 1.0 + jnp.exp(-temp_ref[...])
    out_ref[...] = logits_ref[...] * inv


def _kernel_tc(logits, temp):
    B, V = logits.shape
    BV = 57344
    grid = (pl.cdiv(V, BV),)
    return pl.pallas_call(
        _scale_body,
        grid=grid,
        in_specs=[
            pl.BlockSpec((B, BV), lambda i: (0, i)),
            pl.BlockSpec((B, 1), lambda i: (0, 0)),
        ],
        out_specs=pl.BlockSpec((B, BV), lambda i: (0, i)),
        out_shape=jax.ShapeDtypeStruct((B, V), logits.dtype),
    )(logits, temp)


_NC, _NS, _L = 2, 16, 16
_CW = 6400
_NFULL = 156
_TW = 1664
_UNROLL = 8


def _sc_compute_slab(in_v, out_v, scales, width):
    for rr in range(8):
        sc = scales[rr]

        def inner(jo, carry, rr=rr, sc=sc):
            base = jo * (_L * _UNROLL)
            for u in range(_UNROLL):
                o = base + u * _L
                out_v[rr, pl.ds(o, _L)] = in_v[rr, pl.ds(o, _L)] * sc
            return carry

        lax.fori_loop(0, width // (_L * _UNROLL), inner, 0)


def _sc_body(logits_hbm, temp_hbm, out_hbm, temp_v, in_v, out_v):
    c = lax.axis_index("c")
    s = lax.axis_index("s")
    wid = s * _NC + c
    g = wid % 8
    q = wid // 8
    r0 = pl.multiple_of(g * 8, 8)
    pltpu.sync_copy(temp_hbm, temp_v)

    scales = []
    for rr in range(8):
        off = pl.multiple_of((g * 8 + rr) * _L, _L)
        tv = temp_v[pl.ds(off, _L)]
        scales.append(1.0 + jnp.exp(-tv))

    def do_slab(k, carry):
        off = pl.multiple_of((q + 4 * k) * _CW, 128)
        pltpu.sync_copy(logits_hbm.at[pl.ds(r0, 8), pl.ds(off, _CW)], in_v)
        _sc_compute_slab(in_v, out_v, scales, _CW)
        pltpu.sync_copy(out_v, out_hbm.at[pl.ds(r0, 8), pl.ds(off, _CW)])
        return carry

    lax.fori_loop(0, _NFULL // 4, do_slab, 0)

    @pl.when(q == 3)
    def _tail():
        off = pl.multiple_of(_NFULL * _CW, 128)
        src = logits_hbm.at[pl.ds(r0, 8), pl.ds(off, _TW)]
        pltpu.sync_copy(src, in_v.at[:, pl.ds(0, _TW)])
        _sc_compute_slab(in_v, out_v, scales, _TW)
        pltpu.sync_copy(out_v.at[:, pl.ds(0, _TW)],
                        out_hbm.at[pl.ds(r0, 8), pl.ds(off, _TW)])


def _kernel_sc(logits, temp):
    B, V = logits.shape
    f = pl.kernel(
        _sc_body,
        out_type=jax.ShapeDtypeStruct((B, V), logits.dtype),
        mesh=plsc.VectorSubcoreMesh(core_axis_name="c", subcore_axis_name="s"),
        scratch_types=[
            pltpu.VMEM((B * _L,), jnp.float32),
            pltpu.VMEM((8, _CW), jnp.float32),
            pltpu.VMEM((8, _CW), jnp.float32),
        ],
    )
    temp_rep = jnp.repeat(temp.reshape(B), _L)
    return f(logits, temp_rep)


def kernel(logits, temp):
    return _kernel_sc(logits, temp)


# speedup vs baseline: 1.7890x; 1.7890x over previous
import jax
import jax.numpy as jnp
from jax import lax
from jax.experimental import pallas as pl
from jax.experimental.pallas import tpu as pltpu
from jax.experimental.pallas import tpu_sc as plsc


def _scale_body(logits_ref, temp_ref, out_ref):
    inv = 1.0 + jnp.exp(-temp_ref[...])
    out_ref[...] = logits_ref[...] * inv


def _kernel_tc(logits, temp):
    B, V = logits.shape
    BR, BV = 8, 131072
    grid = (B // BR, pl.cdiv(V, BV))
    return pl.pallas_call(
        _scale_body,
        grid=grid,
        in_specs=[
            pl.BlockSpec((BR, BV), lambda i, j: (i, j)),
            pl.BlockSpec((BR, 1), lambda i, j: (i, 0)),
        ],
        out_specs=pl.BlockSpec((BR, BV), lambda i, j: (i, j)),
        out_shape=jax.ShapeDtypeStruct((B, V), logits.dtype),
    )(logits, temp)


_NC, _NS, _L = 2, 16, 16
_CW = 6400
_NFULL = 156
_TW = 1664
_UNROLL = 8


def _sc_compute_slab(in_v, out_v, scales, width):
    for rr in range(8):
        sc = scales[rr]

        def inner(jo, carry, rr=rr, sc=sc):
            base = jo * (_L * _UNROLL)
            for u in range(_UNROLL):
                o = base + u * _L
                out_v[rr, pl.ds(o, _L)] = in_v[rr, pl.ds(o, _L)] * sc
            return carry

        lax.fori_loop(0, width // (_L * _UNROLL), inner, 0)


def _sc_body(logits_hbm, temp_hbm, out_hbm, temp_v, in_v, out_v):
    c = lax.axis_index("c")
    s = lax.axis_index("s")
    wid = s * _NC + c
    g = wid % 8
    q = wid // 8
    r0 = pl.multiple_of(g * 8, 8)
    pltpu.sync_copy(temp_hbm, temp_v)

    scales = []
    for rr in range(8):
        off = pl.multiple_of((g * 8 + rr) * _L, _L)
        tv = temp_v[pl.ds(off, _L)]
        scales.append(1.0 + jnp.exp(-tv))

    def do_slab(k, carry):
        off = pl.multiple_of((q + 4 * k) * _CW, 128)
        pltpu.sync_copy(logits_hbm.at[pl.ds(r0, 8), pl.ds(off, _CW)], in_v)
        _sc_compute_slab(in_v, out_v, scales, _CW)
        pltpu.sync_copy(out_v, out_hbm.at[pl.ds(r0, 8), pl.ds(off, _CW)])
        return carry

    lax.fori_loop(0, _NFULL // 4, do_slab, 0)

    @pl.when(q == 3)
    def _tail():
        off = pl.multiple_of(_NFULL * _CW, 128)
        src = logits_hbm.at[pl.ds(r0, 8), pl.ds(off, _TW)]
        pltpu.sync_copy(src, in_v.at[:, pl.ds(0, _TW)])
        _sc_compute_slab(in_v, out_v, scales, _TW)
        pltpu.sync_copy(out_v.at[:, pl.ds(0, _TW)],
                        out_hbm.at[pl.ds(r0, 8), pl.ds(off, _TW)])


def _kernel_sc(logits, temp):
    B, V = logits.shape
    f = pl.kernel(
        _sc_body,
        out_type=jax.ShapeDtypeStruct((B, V), logits.dtype),
        mesh=plsc.VectorSubcoreMesh(core_axis_name="c", subcore_axis_name="s"),
        scratch_types=[
            pltpu.VMEM((B * _L,), jnp.float32),
            pltpu.VMEM((8, _CW), jnp.float32),
            pltpu.VMEM((8, _CW), jnp.float32),
        ],
    )
    temp_rep = jnp.repeat(temp.reshape(B), _L)
    return f(logits, temp_rep)


def kernel(logits, temp):
    return _kernel_tc(logits, temp)


# speedup vs baseline: 1.8216x; 1.0182x over previous
import jax
import jax.numpy as jnp
from jax import lax
from jax.experimental import pallas as pl
from jax.experimental.pallas import tpu as pltpu
from jax.experimental.pallas import tpu_sc as plsc


def _scale_body(logits_ref, temp_ref, out_ref):
    inv = 1.0 + jnp.exp(-temp_ref[...])
    out_ref[...] = logits_ref[...] * inv


def _kernel_tc(logits, temp):
    B, V = logits.shape
    BV = 62336
    grid = (pl.cdiv(V, BV),)
    return pl.pallas_call(
        _scale_body,
        grid=grid,
        in_specs=[
            pl.BlockSpec((B, BV), lambda i: (0, i)),
            pl.BlockSpec((B, 1), lambda i: (0, 0)),
        ],
        out_specs=pl.BlockSpec((B, BV), lambda i: (0, i)),
        out_shape=jax.ShapeDtypeStruct((B, V), logits.dtype),
        compiler_params=pltpu.CompilerParams(vmem_limit_bytes=128 * 1024 * 1024),
    )(logits, temp)


_NC, _NS, _L = 2, 16, 16
_CW = 6400
_NFULL = 156
_TW = 1664
_UNROLL = 8


def _sc_compute_slab(in_v, out_v, scales, width):
    for rr in range(8):
        sc = scales[rr]

        def inner(jo, carry, rr=rr, sc=sc):
            base = jo * (_L * _UNROLL)
            for u in range(_UNROLL):
                o = base + u * _L
                out_v[rr, pl.ds(o, _L)] = in_v[rr, pl.ds(o, _L)] * sc
            return carry

        lax.fori_loop(0, width // (_L * _UNROLL), inner, 0)


def _sc_body(logits_hbm, temp_hbm, out_hbm, temp_v, in_v, out_v):
    c = lax.axis_index("c")
    s = lax.axis_index("s")
    wid = s * _NC + c
    g = wid % 8
    q = wid // 8
    r0 = pl.multiple_of(g * 8, 8)
    pltpu.sync_copy(temp_hbm, temp_v)

    scales = []
    for rr in range(8):
        off = pl.multiple_of((g * 8 + rr) * _L, _L)
        tv = temp_v[pl.ds(off, _L)]
        scales.append(1.0 + jnp.exp(-tv))

    def do_slab(k, carry):
        off = pl.multiple_of((q + 4 * k) * _CW, 128)
        pltpu.sync_copy(logits_hbm.at[pl.ds(r0, 8), pl.ds(off, _CW)], in_v)
        _sc_compute_slab(in_v, out_v, scales, _CW)
        pltpu.sync_copy(out_v, out_hbm.at[pl.ds(r0, 8), pl.ds(off, _CW)])
        return carry

    lax.fori_loop(0, _NFULL // 4, do_slab, 0)

    @pl.when(q == 3)
    def _tail():
        off = pl.multiple_of(_NFULL * _CW, 128)
        src = logits_hbm.at[pl.ds(r0, 8), pl.ds(off, _TW)]
        pltpu.sync_copy(src, in_v.at[:, pl.ds(0, _TW)])
        _sc_compute_slab(in_v, out_v, scales, _TW)
        pltpu.sync_copy(out_v.at[:, pl.ds(0, _TW)],
                        out_hbm.at[pl.ds(r0, 8), pl.ds(off, _TW)])


def _kernel_sc(logits, temp):
    B, V = logits.shape
    f = pl.kernel(
        _sc_body,
        out_type=jax.ShapeDtypeStruct((B, V), logits.dtype),
        mesh=plsc.VectorSubcoreMesh(core_axis_name="c", subcore_axis_name="s"),
        scratch_types=[
            pltpu.VMEM((B * _L,), jnp.float32),
            pltpu.VMEM((8, _CW), jnp.float32),
            pltpu.VMEM((8, _CW), jnp.float32),
        ],
    )
    temp_rep = jnp.repeat(temp.reshape(B), _L)
    return f(logits, temp_rep)


def kernel(logits, temp):
    return _kernel_tc(logits, temp)


# speedup vs baseline: 1.8333x; 1.0064x over previous
import jax
import jax.numpy as jnp
from jax import lax
from jax.experimental import pallas as pl
from jax.experimental.pallas import tpu as pltpu
from jax.experimental.pallas import tpu_sc as plsc


def _scale_body(logits_ref, temp_ref, out_ref):
    inv = 1.0 + jnp.exp(-temp_ref[...])
    out_ref[...] = logits_ref[...] * inv


def _kernel_tc(logits, temp):
    B, V = logits.shape
    BV = 57344
    grid = (pl.cdiv(V, BV),)
    return pl.pallas_call(
        _scale_body,
        grid=grid,
        in_specs=[
            pl.BlockSpec((B, BV), lambda i: (0, i)),
            pl.BlockSpec((B, 1), lambda i: (0, 0)),
        ],
        out_specs=pl.BlockSpec((B, BV), lambda i: (0, i)),
        out_shape=jax.ShapeDtypeStruct((B, V), logits.dtype),
        compiler_params=pltpu.CompilerParams(vmem_limit_bytes=128 * 1024 * 1024),
    )(logits, temp)


_NC, _NS, _L = 2, 16, 16
_CW = 6400
_NFULL = 156
_TW = 1664
_UNROLL = 8


def _sc_compute_slab(in_v, out_v, scales, width):
    for rr in range(8):
        sc = scales[rr]

        def inner(jo, carry, rr=rr, sc=sc):
            base = jo * (_L * _UNROLL)
            for u in range(_UNROLL):
                o = base + u * _L
                out_v[rr, pl.ds(o, _L)] = in_v[rr, pl.ds(o, _L)] * sc
            return carry

        lax.fori_loop(0, width // (_L * _UNROLL), inner, 0)


def _sc_body(logits_hbm, temp_hbm, out_hbm, temp_v, in_v, out_v):
    c = lax.axis_index("c")
    s = lax.axis_index("s")
    wid = s * _NC + c
    g = wid % 8
    q = wid // 8
    r0 = pl.multiple_of(g * 8, 8)
    pltpu.sync_copy(temp_hbm, temp_v)

    scales = []
    for rr in range(8):
        off = pl.multiple_of((g * 8 + rr) * _L, _L)
        tv = temp_v[pl.ds(off, _L)]
        scales.append(1.0 + jnp.exp(-tv))

    def do_slab(k, carry):
        off = pl.multiple_of((q + 4 * k) * _CW, 128)
        pltpu.sync_copy(logits_hbm.at[pl.ds(r0, 8), pl.ds(off, _CW)], in_v)
        _sc_compute_slab(in_v, out_v, scales, _CW)
        pltpu.sync_copy(out_v, out_hbm.at[pl.ds(r0, 8), pl.ds(off, _CW)])
        return carry

    lax.fori_loop(0, _NFULL // 4, do_slab, 0)

    @pl.when(q == 3)
    def _tail():
        off = pl.multiple_of(_NFULL * _CW, 128)
        src = logits_hbm.at[pl.ds(r0, 8), pl.ds(off, _TW)]
        pltpu.sync_copy(src, in_v.at[:, pl.ds(0, _TW)])
        _sc_compute_slab(in_v, out_v, scales, _TW)
        pltpu.sync_copy(out_v.at[:, pl.ds(0, _TW)],
                        out_hbm.at[pl.ds(r0, 8), pl.ds(off, _TW)])


def _kernel_sc(logits, temp):
    B, V = logits.shape
    f = pl.kernel(
        _sc_body,
        out_type=jax.ShapeDtypeStruct((B, V), logits.dtype),
        mesh=plsc.VectorSubcoreMesh(core_axis_name="c", subcore_axis_name="s"),
        scratch_types=[
            pltpu.VMEM((B * _L,), jnp.float32),
            pltpu.VMEM((8, _CW), jnp.float32),
            pltpu.VMEM((8, _CW), jnp.float32),
        ],
    )
    temp_rep = jnp.repeat(temp.reshape(B), _L)
    return f(logits, temp_rep)


def kernel(logits, temp):
    return _kernel_tc(logits, temp)
